# baseline (speedup 1.0000x reference)
import functools

import jax
import jax.numpy as jnp
from jax import lax
from jax.experimental import pallas as pl
from jax.experimental.pallas import tpu as pltpu
from jax.experimental.pallas import tpu_sc as plsc

VOCAB = 100000
EMBED = 128
BATCH = 4096
HIST = 50
B = BATCH * HIST

NC = 2
NS = 16
NW = NC * NS
B_PER_W = B // NW
CHUNK = 128
NCHUNK = B_PER_W // CHUNK
NBUF = 5
NGROUP = NCHUNK // NBUF

_mesh = plsc.VectorSubcoreMesh(core_axis_name="c", subcore_axis_name="s")


@functools.partial(
    pl.kernel,
    out_type=jax.ShapeDtypeStruct((B, EMBED), jnp.float32),
    mesh=_mesh,
    scratch_types=[
        pltpu.VMEM((NCHUNK, CHUNK), jnp.int32),
        pltpu.VMEM((NCHUNK, CHUNK), jnp.int32),
        pltpu.VMEM((NBUF, CHUNK, EMBED), jnp.float32),
        pltpu.SemaphoreType.DMA,
        pltpu.SemaphoreType.DMA((NBUF,)),
        pltpu.SemaphoreType.DMA((NBUF,)),
    ],
)
def _sc_gather(idx_hbm, dst_hbm, table_hbm, out_hbm, idx_v, dst_v, rows_v,
               ssem, gsem, osem):
    wid = lax.axis_index("s") * NC + lax.axis_index("c")
    s1 = pltpu.async_copy(idx_hbm.at[wid], idx_v, ssem)
    s2 = pltpu.async_copy(dst_hbm.at[wid], dst_v, ssem)
    s1.wait()
    s2.wait()

    def fire_gather(c, b):
        pltpu.async_copy(table_hbm.at[idx_v.at[c]], rows_v.at[b], gsem.at[b])

    def wait_gather(c, b):
        pltpu.make_async_copy(
            table_hbm.at[idx_v.at[c]], rows_v.at[b], gsem.at[b]
        ).wait()

    def fire_scatter(c, b):
        pltpu.async_copy(rows_v.at[b], out_hbm.at[dst_v.at[c]], osem.at[b])

    def wait_scatter(c, b):
        pltpu.make_async_copy(
            rows_v.at[b], out_hbm.at[dst_v.at[c]], osem.at[b]
        ).wait()

    for b in range(NBUF):
        fire_gather(b, b)

    wait_gather(0, 0)
    fire_scatter(0, 0)
    for b in range(1, NBUF):
        c = b
        wait_gather(c, b)
        fire_scatter(c, b)
        wait_scatter(c - 1, b - 1)
        fire_gather(c - 1 + NBUF, b - 1)

    @pl.loop(1, NGROUP - 1)
    def _group(g):
        c0 = g * NBUF
        for b in range(NBUF):
            c = c0 + b
            pb = (b - 1) % NBUF
            wait_gather(c, b)
            fire_scatter(c, b)
            wait_scatter(c - 1, pb)
            fire_gather(c - 1 + NBUF, pb)

    c0 = (NGROUP - 1) * NBUF
    for b in range(NBUF):
        c = c0 + b
        pb = (b - 1) % NBUF
        wait_gather(c, b)
        fire_scatter(c, b)
        wait_scatter(c - 1, pb)
        if c - 1 + NBUF < NCHUNK:
            fire_gather(c - 1 + NBUF, pb)
    wait_scatter(NCHUNK - 1, (NCHUNK - 1) % NBUF)


def kernel(x, table):
    idx = x.reshape(NW, NCHUNK, CHUNK).astype(jnp.int32)
    p = jnp.arange(B, dtype=jnp.int32)
    dst = ((p % HIST) * BATCH + p // HIST).reshape(NW, NCHUNK, CHUNK)
    out = _sc_gather(idx, dst, table)
    return out.reshape(HIST, BATCH, EMBED).swapaxes(0, 1)


# speedup vs baseline: 1.0037x; 1.0037x over previous
import functools

import jax
import jax.numpy as jnp
import numpy as np
from jax import lax
from jax.experimental import pallas as pl
from jax.experimental.pallas import tpu as pltpu
from jax.experimental.pallas import tpu_sc as plsc

VOCAB = 100000
EMBED = 128
BATCH = 4096
HIST = 50
B = BATCH * HIST

NC = 2
NS = 16
NW = NC * NS
B_PER_W = B // NW
CHUNK = 128
NCHUNK = B_PER_W // CHUNK
NBUF = 5
NGROUP = NCHUNK // NBUF

_mesh = plsc.VectorSubcoreMesh(core_axis_name="c", subcore_axis_name="s")


@functools.partial(
    pl.kernel,
    out_type=jax.ShapeDtypeStruct((B, EMBED), jnp.float32),
    mesh=_mesh,
    scratch_types=[
        pltpu.VMEM((NCHUNK, CHUNK), jnp.int32),
        pltpu.VMEM((NCHUNK, CHUNK), jnp.int32),
        pltpu.VMEM((NBUF, CHUNK, EMBED), jnp.float32),
        pltpu.SemaphoreType.DMA,
        pltpu.SemaphoreType.DMA((NBUF,)),
        pltpu.SemaphoreType.DMA((NBUF,)),
    ],
)
def _sc_gather(idx_hbm, dst_hbm, table_hbm, out_hbm, idx_v, dst_v, rows_v,
               ssem, gsem, osem):
    wid = lax.axis_index("s") * NC + lax.axis_index("c")
    s1 = pltpu.async_copy(idx_hbm.at[wid], idx_v, ssem)
    s2 = pltpu.async_copy(dst_hbm.at[wid], dst_v, ssem)
    s1.wait()
    s2.wait()

    def fire_gather(c, b):
        pltpu.async_copy(table_hbm.at[idx_v.at[c]], rows_v.at[b], gsem.at[b])

    def wait_gather(c, b):
        pltpu.make_async_copy(
            table_hbm.at[idx_v.at[c]], rows_v.at[b], gsem.at[b]
        ).wait()

    def fire_scatter(c, b):
        pltpu.async_copy(rows_v.at[b], out_hbm.at[dst_v.at[c]], osem.at[b])

    def wait_scatter(c, b):
        pltpu.make_async_copy(
            rows_v.at[b], out_hbm.at[dst_v.at[c]], osem.at[b]
        ).wait()

    for b in range(NBUF):
        fire_gather(b, b)

    wait_gather(0, 0)
    fire_scatter(0, 0)
    for b in range(1, NBUF):
        c = b
        wait_gather(c, b)
        fire_scatter(c, b)
        wait_scatter(c - 1, b - 1)
        fire_gather(c - 1 + NBUF, b - 1)

    @pl.loop(1, NGROUP - 1)
    def _group(g):
        c0 = g * NBUF
        for b in range(NBUF):
            c = c0 + b
            pb = (b - 1) % NBUF
            wait_gather(c, b)
            fire_scatter(c, b)
            wait_scatter(c - 1, pb)
            fire_gather(c - 1 + NBUF, pb)

    c0 = (NGROUP - 1) * NBUF
    for b in range(NBUF):
        c = c0 + b
        pb = (b - 1) % NBUF
        wait_gather(c, b)
        fire_scatter(c, b)
        wait_scatter(c - 1, pb)
        if c - 1 + NBUF < NCHUNK:
            fire_gather(c - 1 + NBUF, pb)
    wait_scatter(NCHUNK - 1, (NCHUNK - 1) % NBUF)


_P = np.arange(B, dtype=np.int32)
_DST = ((_P % HIST) * BATCH + _P // HIST).reshape(NW, NCHUNK, CHUNK)


def kernel(x, table):
    idx = x.astype(jnp.int32).reshape(NW, NCHUNK, CHUNK)
    out = _sc_gather(idx, jnp.asarray(_DST), table)
    return out.reshape(HIST, BATCH, EMBED).swapaxes(0, 1)


# speedup vs baseline: 1.0248x; 1.0209x over previous
import functools

import jax
import jax.numpy as jnp
from jax import lax
from jax.experimental import pallas as pl
from jax.experimental.pallas import tpu as pltpu
from jax.experimental.pallas import tpu_sc as plsc

VOCAB = 100000
EMBED = 128
BATCH = 4096
HIST = 50
B = BATCH * HIST

NC = 2
NS = 16
NW = NC * NS
B_PER_W = B // NW
CHUNK = 128
NCHUNK = B_PER_W // CHUNK
NBUF = 5
NGROUP = NCHUNK // NBUF

_mesh = plsc.VectorSubcoreMesh(core_axis_name="c", subcore_axis_name="s")


@functools.partial(
    pl.kernel,
    out_type=jax.ShapeDtypeStruct((B, EMBED), jnp.float32),
    mesh=_mesh,
    scratch_types=[
        pltpu.VMEM((NCHUNK, CHUNK), jnp.int32),
        pltpu.VMEM((NBUF, CHUNK, EMBED), jnp.float32),
        pltpu.SemaphoreType.DMA,
        pltpu.SemaphoreType.DMA((NBUF,)),
        pltpu.SemaphoreType.DMA((NBUF,)),
    ],
)
def _sc_gather(idx_hbm, table_hbm, out_hbm, idx_v, rows_v, ssem, gsem, osem):
    wid = lax.axis_index("s") * NC + lax.axis_index("c")
    base = wid * B_PER_W
    pltpu.async_copy(idx_hbm.at[wid], idx_v, ssem).wait()

    def fire_gather(c, b):
        pltpu.async_copy(table_hbm.at[idx_v.at[c]], rows_v.at[b], gsem.at[b])

    def wait_gather(c, b):
        pltpu.make_async_copy(
            table_hbm.at[idx_v.at[c]], rows_v.at[b], gsem.at[b]
        ).wait()

    def fire_write(c, b):
        pltpu.async_copy(
            rows_v.at[b], out_hbm.at[pl.ds(base + c * CHUNK, CHUNK)],
            osem.at[b],
        )

    def wait_write(c, b):
        pltpu.make_async_copy(
            rows_v.at[b], out_hbm.at[pl.ds(base + c * CHUNK, CHUNK)],
            osem.at[b],
        ).wait()

    for b in range(NBUF):
        fire_gather(b, b)

    wait_gather(0, 0)
    fire_write(0, 0)
    for b in range(1, NBUF):
        c = b
        wait_gather(c, b)
        fire_write(c, b)
        wait_write(c - 1, b - 1)
        fire_gather(c - 1 + NBUF, b - 1)

    @pl.loop(1, NGROUP - 1)
    def _group(g):
        c0 = g * NBUF
        for b in range(NBUF):
            c = c0 + b
            pb = (b - 1) % NBUF
            wait_gather(c, b)
            fire_write(c, b)
            wait_write(c - 1, pb)
            fire_gather(c - 1 + NBUF, pb)

    c0 = (NGROUP - 1) * NBUF
    for b in range(NBUF):
        c = c0 + b
        pb = (b - 1) % NBUF
        wait_gather(c, b)
        fire_write(c, b)
        wait_write(c - 1, pb)
        if c - 1 + NBUF < NCHUNK:
            fire_gather(c - 1 + NBUF, pb)
    wait_write(NCHUNK - 1, (NCHUNK - 1) % NBUF)


def kernel(x, table):
    idx = x.T.astype(jnp.int32).reshape(NW, NCHUNK, CHUNK)
    out = _sc_gather(idx, table)
    return out.reshape(HIST, BATCH, EMBED).swapaxes(0, 1)
